# baseline (speedup 1.0000x reference)
import functools

import jax
import jax.numpy as jnp
from jax.experimental import pallas as pl
from jax.experimental.pallas import tpu as pltpu
from jax.experimental.pallas import tpu_sc as plsc

NUM_FIELDS = 26
FIELD_DIM = 100000
EMBED_DIM = 16
ATT_T = 4
BATCH = 4096
NUM_IDX = BATCH * NUM_FIELDS
NW = 32
BPW = NUM_IDX // NW
BBLK = 512


def _sc_gather_e1(e1flat, idx):
    mesh = plsc.VectorSubcoreMesh(core_axis_name="c", subcore_axis_name="s")

    @functools.partial(
        pl.kernel,
        out_type=jax.ShapeDtypeStruct((NUM_IDX,), jnp.float32),
        mesh=mesh,
        scratch_types=[
            pltpu.VMEM((BPW,), jnp.int32),
            pltpu.VMEM((BPW,), jnp.float32),
            pltpu.SemaphoreType.DMA,
        ],
    )
    def k(t_hbm, i_hbm, o_hbm, idx_v, val_v, sem):
        wid = jax.lax.axis_index("s") * 2 + jax.lax.axis_index("c")
        base = wid * BPW
        pltpu.sync_copy(i_hbm.at[pl.ds(base, BPW)], idx_v)
        pltpu.async_copy(t_hbm.at[idx_v], val_v, sem).wait()
        pltpu.sync_copy(val_v, o_hbm.at[pl.ds(base, BPW)])

    return k(e1flat, idx)


def _tc_body(*refs):
    e2_refs = refs[:NUM_FIELDS]
    e1_ref = refs[NUM_FIELDS]
    w0_ref, w1_ref, b1_ref, w2_ref, p_ref, o_ref = refs[NUM_FIELDS + 1 :]
    e = [r[...] for r in e2_refs]
    w1v = w1_ref[...]
    b1v = b1_ref[...]
    w2v = w2_ref[...]
    pv = p_ref[...]
    bf = jnp.bfloat16
    w5T = jnp.concatenate([w1v, pv], axis=1).T.astype(bf)
    b1c = b1v.reshape(ATT_T, 1)
    logit_rows = []
    s_rows = []
    for a in range(NUM_FIELDS):
        for b in range(a + 1, NUM_FIELDS):
            ip = (e[a] * e[b]).astype(bf)
            h5 = jax.lax.dot(w5T, ip, preferred_element_type=jnp.float32)
            lt = jnp.maximum(h5[:ATT_T] + b1c, 0.0) * w2v
            logit_rows.append(lt.sum(axis=0, keepdims=True))
            s_rows.append(h5[ATT_T:])
    logits = jnp.concatenate(logit_rows, axis=0)
    S = jnp.concatenate(s_rows, axis=0)
    mx = logits.max(axis=0, keepdims=True)
    ex = jnp.exp(logits - mx)
    s = ex.sum(axis=0, keepdims=True)
    att_part = (ex * S).sum(axis=0, keepdims=True) / s
    e1s = e1_ref[...].sum(axis=0, keepdims=True)
    o_ref[...] = jax.nn.sigmoid(w0_ref[...] + e1s + att_part)


def _tc_compute(e2gT, e1g2, w0, w1, b1, w2, p):
    nsteps = BATCH // BBLK
    e2_specs = [
        pl.BlockSpec((EMBED_DIM, BBLK), lambda i, f=f: (0, f * nsteps + i))
        for f in range(NUM_FIELDS)
    ]
    e1_specs = [pl.BlockSpec((NUM_FIELDS, BBLK), lambda i: (0, i))]
    par_specs = [
        pl.BlockSpec((1, 1), lambda i: (0, 0)),
        pl.BlockSpec((EMBED_DIM, ATT_T), lambda i: (0, 0)),
        pl.BlockSpec((1, ATT_T), lambda i: (0, 0)),
        pl.BlockSpec((ATT_T, 1), lambda i: (0, 0)),
        pl.BlockSpec((EMBED_DIM, 1), lambda i: (0, 0)),
    ]
    return pl.pallas_call(
        _tc_body,
        grid=(nsteps,),
        in_specs=e2_specs + e1_specs + par_specs,
        out_specs=pl.BlockSpec((1, BBLK), lambda i: (0, i)),
        out_shape=jax.ShapeDtypeStruct((1, BATCH), jnp.float32),
    )(*([e2gT] * NUM_FIELDS + [e1g2] + [w0, w1, b1, w2, p]))


def kernel(x, w0, embed1_table, embed2_table, att_w1, att_b1, att_w2, p):
    offsets = jnp.arange(NUM_FIELDS, dtype=x.dtype) * FIELD_DIM
    xo2 = x.T + offsets[:, None]
    e1g = _sc_gather_e1(embed1_table.reshape(-1), xo2.reshape(-1))
    e1g2 = e1g.reshape(NUM_FIELDS, BATCH)
    e2g = embed2_table.at[xo2.reshape(-1)].get(mode="promise_in_bounds")
    e2gT = e2g.T
    out = _tc_compute(
        e2gT,
        e1g2,
        w0.reshape(1, 1),
        att_w1,
        att_b1.reshape(1, ATT_T),
        att_w2,
        p.reshape(EMBED_DIM, 1),
    )
    return out.reshape(BATCH, 1)


# speedup vs baseline: 1.3289x; 1.3289x over previous
import functools

import jax
import jax.numpy as jnp
from jax.experimental import pallas as pl
from jax.experimental.pallas import tpu as pltpu
from jax.experimental.pallas import tpu_sc as plsc

NUM_FIELDS = 26
FIELD_DIM = 100000
EMBED_DIM = 16
ATT_T = 4
BATCH = 4096
NUM_IDX = BATCH * NUM_FIELDS
NW = 32
BPW = NUM_IDX // NW
BBLK = 512


def _sc_gather_e1(e1flat, idx):
    mesh = plsc.VectorSubcoreMesh(core_axis_name="c", subcore_axis_name="s")

    @functools.partial(
        pl.kernel,
        out_type=jax.ShapeDtypeStruct((NUM_IDX,), jnp.float32),
        mesh=mesh,
        scratch_types=[
            pltpu.VMEM((BPW,), jnp.int32),
            pltpu.VMEM((BPW,), jnp.float32),
            pltpu.SemaphoreType.DMA,
        ],
    )
    def k(t_hbm, i_hbm, o_hbm, idx_v, val_v, sem):
        wid = jax.lax.axis_index("s") * 2 + jax.lax.axis_index("c")
        base = wid * BPW
        pltpu.sync_copy(i_hbm.at[pl.ds(base, BPW)], idx_v)
        pltpu.async_copy(t_hbm.at[idx_v], val_v, sem).wait()
        pltpu.sync_copy(val_v, o_hbm.at[pl.ds(base, BPW)])

    return k(e1flat, idx)


def _tc_body(*refs):
    e2_refs = refs[:NUM_FIELDS]
    e1_ref = refs[NUM_FIELDS]
    w0_ref, w1_ref, b1_ref, w2_ref, p_ref, o_ref = refs[NUM_FIELDS + 1 :]
    e = [r[...] for r in e2_refs]
    w1v = w1_ref[...]
    b1v = b1_ref[...]
    w2v = w2_ref[...]
    pv = p_ref[...]
    bf = jnp.bfloat16
    w5T = jnp.concatenate([w1v, pv], axis=1).T.astype(bf)
    b1c = b1v.reshape(ATT_T, 1)
    logit_rows = []
    s_rows = []
    for a in range(NUM_FIELDS):
        for b in range(a + 1, NUM_FIELDS):
            ip = (e[a] * e[b]).astype(bf)
            h5 = jax.lax.dot(w5T, ip, preferred_element_type=jnp.float32)
            lt = jnp.maximum(h5[:ATT_T] + b1c, 0.0) * w2v
            logit_rows.append(lt.sum(axis=0, keepdims=True))
            s_rows.append(h5[ATT_T:])
    logits = jnp.concatenate(logit_rows, axis=0)
    S = jnp.concatenate(s_rows, axis=0)
    mx = logits.max(axis=0, keepdims=True)
    ex = jnp.exp(logits - mx)
    s = ex.sum(axis=0, keepdims=True)
    att_part = (ex * S).sum(axis=0, keepdims=True) / s
    e1s = e1_ref[...].sum(axis=0, keepdims=True)
    o_ref[...] = jax.nn.sigmoid(w0_ref[...] + e1s + att_part)


def _tc_compute(e2gT, e1g2, w0, w1, b1, w2, p):
    nsteps = BATCH // BBLK
    e2_specs = [
        pl.BlockSpec((EMBED_DIM, BBLK), lambda i, f=f: (0, f * nsteps + i))
        for f in range(NUM_FIELDS)
    ]
    e1_specs = [pl.BlockSpec((NUM_FIELDS, BBLK), lambda i: (0, i))]
    par_specs = [
        pl.BlockSpec((1, 1), lambda i: (0, 0)),
        pl.BlockSpec((EMBED_DIM, ATT_T), lambda i: (0, 0)),
        pl.BlockSpec((1, ATT_T), lambda i: (0, 0)),
        pl.BlockSpec((ATT_T, 1), lambda i: (0, 0)),
        pl.BlockSpec((EMBED_DIM, 1), lambda i: (0, 0)),
    ]
    return pl.pallas_call(
        _tc_body,
        grid=(nsteps,),
        in_specs=e2_specs + e1_specs + par_specs,
        out_specs=pl.BlockSpec((1, BBLK), lambda i: (0, i)),
        out_shape=jax.ShapeDtypeStruct((1, BATCH), jnp.float32),
    )(*([e2gT] * NUM_FIELDS + [e1g2] + [w0, w1, b1, w2, p]))


def kernel(x, w0, embed1_table, embed2_table, att_w1, att_b1, att_w2, p):
    offsets = jnp.arange(NUM_FIELDS, dtype=x.dtype) * FIELD_DIM
    xo2 = x.T + offsets[:, None]
    e1g = _sc_gather_e1(embed1_table.reshape(-1), xo2.reshape(-1))
    e1g2 = e1g.reshape(NUM_FIELDS, BATCH)
    e2g = jnp.take(embed2_table, xo2.reshape(-1), axis=0)
    e2gT = e2g.T
    out = _tc_compute(
        e2gT,
        e1g2,
        w0.reshape(1, 1),
        att_w1,
        att_b1.reshape(1, ATT_T),
        att_w2,
        p.reshape(EMBED_DIM, 1),
    )
    return out.reshape(BATCH, 1)


# speedup vs baseline: 1.3293x; 1.0002x over previous
import functools

import jax
import jax.numpy as jnp
from jax.experimental import pallas as pl
from jax.experimental.pallas import tpu as pltpu
from jax.experimental.pallas import tpu_sc as plsc

NUM_FIELDS = 26
FIELD_DIM = 100000
EMBED_DIM = 16
ATT_T = 4
BATCH = 4096
NUM_IDX = BATCH * NUM_FIELDS
NW = 32
BPW = NUM_IDX // NW
BBLK = 512


def _sc_gather_e1(e1flat, idx):
    mesh = plsc.VectorSubcoreMesh(core_axis_name="c", subcore_axis_name="s")

    @functools.partial(
        pl.kernel,
        out_type=jax.ShapeDtypeStruct((NUM_IDX,), jnp.float32),
        mesh=mesh,
        scratch_types=[
            pltpu.VMEM((BPW,), jnp.int32),
            pltpu.VMEM((BPW,), jnp.float32),
            pltpu.SemaphoreType.DMA,
        ],
    )
    def k(t_hbm, i_hbm, o_hbm, idx_v, val_v, sem):
        wid = jax.lax.axis_index("s") * 2 + jax.lax.axis_index("c")
        base = wid * BPW
        pltpu.sync_copy(i_hbm.at[pl.ds(base, BPW)], idx_v)
        pltpu.async_copy(t_hbm.at[idx_v], val_v, sem).wait()
        pltpu.sync_copy(val_v, o_hbm.at[pl.ds(base, BPW)])

    return k(e1flat, idx)


def _tc_body(*refs):
    e2_refs = refs[:NUM_FIELDS]
    e1_ref = refs[NUM_FIELDS]
    w0_ref, w1_ref, b1_ref, w2_ref, p_ref, o_ref = refs[NUM_FIELDS + 1 :]
    e = [r[...] for r in e2_refs]
    w1v = w1_ref[...]
    b1v = b1_ref[...]
    w2v = w2_ref[...]
    pv = p_ref[...]
    bf = jnp.bfloat16
    w5T = jnp.concatenate([w1v, pv], axis=1).T.astype(bf)
    b1c = b1v.reshape(ATT_T, 1)
    logit_rows = []
    s_rows = []
    for a in range(NUM_FIELDS):
        for b in range(a + 1, NUM_FIELDS):
            ip = (e[a] * e[b]).astype(bf)
            h5 = jax.lax.dot(w5T, ip, preferred_element_type=jnp.float32)
            lt = jnp.maximum(h5[:ATT_T] + b1c, 0.0) * w2v
            logit_rows.append(lt.sum(axis=0, keepdims=True))
            s_rows.append(h5[ATT_T:])
    logits = jnp.concatenate(logit_rows, axis=0)
    S = jnp.concatenate(s_rows, axis=0)
    mx = logits.max(axis=0, keepdims=True)
    ex = jnp.exp(logits - mx)
    s = ex.sum(axis=0, keepdims=True)
    att_part = (ex * S).sum(axis=0, keepdims=True) / s
    e1s = e1_ref[...].sum(axis=0, keepdims=True)
    o_ref[...] = jax.nn.sigmoid(w0_ref[...] + e1s + att_part)


def _tc_compute(e2gT, e1g2, w0, w1, b1, w2, p):
    nsteps = BATCH // BBLK
    e2_specs = [
        pl.BlockSpec((EMBED_DIM, BBLK), lambda i, f=f: (0, f * nsteps + i))
        for f in range(NUM_FIELDS)
    ]
    e1_specs = [pl.BlockSpec((NUM_FIELDS, BBLK), lambda i: (0, i))]
    par_specs = [
        pl.BlockSpec((1, 1), lambda i: (0, 0)),
        pl.BlockSpec((EMBED_DIM, ATT_T), lambda i: (0, 0)),
        pl.BlockSpec((1, ATT_T), lambda i: (0, 0)),
        pl.BlockSpec((ATT_T, 1), lambda i: (0, 0)),
        pl.BlockSpec((EMBED_DIM, 1), lambda i: (0, 0)),
    ]
    return pl.pallas_call(
        _tc_body,
        grid=(nsteps,),
        in_specs=e2_specs + e1_specs + par_specs,
        out_specs=pl.BlockSpec((1, BBLK), lambda i: (0, i)),
        out_shape=jax.ShapeDtypeStruct((1, BATCH), jnp.float32),
    )(*([e2gT] * NUM_FIELDS + [e1g2] + [w0, w1, b1, w2, p]))


def kernel(x, w0, embed1_table, embed2_table, att_w1, att_b1, att_w2, p):
    offsets = jnp.arange(NUM_FIELDS, dtype=x.dtype) * FIELD_DIM
    xo2 = x.T + offsets[:, None]
    e1lin = embed1_table.reshape(-1) * (1.0 + 0.0 * w0[0])
    e1g = _sc_gather_e1(e1lin, xo2.reshape(-1))
    e1g2 = e1g.reshape(NUM_FIELDS, BATCH)
    e2g = jnp.take(embed2_table, xo2.reshape(-1), axis=0)
    e2gT = e2g.T
    out = _tc_compute(
        e2gT,
        e1g2,
        w0.reshape(1, 1),
        att_w1,
        att_b1.reshape(1, ATT_T),
        att_w2,
        p.reshape(EMBED_DIM, 1),
    )
    return out.reshape(BATCH, 1)
